# baseline (speedup 1.0000x reference)
import functools

import jax
import jax.numpy as jnp
from jax import lax
from jax.experimental import pallas as pl
from jax.experimental.pallas import tpu as pltpu
from jax.experimental.pallas import tpu_sc as plsc

N_CP = 20
CP_STEP = 40.0

NC = 2
NS = 16
NW = NC * NS
LANES = 16


@functools.partial(jax.jit, static_argnames=("b_per_w", "n_items"))
def _trend_sc(t, idx, m_tab, k_tab, d_flat, b_per_w, n_items):
    B = t.shape[0]
    n_groups = b_per_w // LANES
    mesh = plsc.VectorSubcoreMesh(core_axis_name="c", subcore_axis_name="s")

    @functools.partial(
        pl.kernel,
        mesh=mesh,
        compiler_params=pltpu.CompilerParams(
            needs_layout_passes=False, use_tc_tiling_on_sc=False
        ),
        out_type=jax.ShapeDtypeStruct((B,), jnp.float32),
        scratch_types=[
            pltpu.VMEM((b_per_w,), jnp.int32),
            [pltpu.VMEM((b_per_w,), jnp.int32)] * N_CP,
            pltpu.VMEM((N_CP * b_per_w,), jnp.float32),
            pltpu.VMEM((b_per_w,), jnp.float32),
            pltpu.VMEM((b_per_w,), jnp.float32),
            pltpu.VMEM((b_per_w,), jnp.float32),
            pltpu.VMEM((b_per_w,), jnp.float32),
            pltpu.SemaphoreType.DMA,
            [pltpu.SemaphoreType.DMA] * N_CP,
        ],
    )
    def sc_kernel(t_hbm, idx_hbm, m_hbm, k_hbm, d_hbm, out_hbm,
                  id_v, jx_vs, d_v, t_v, m_v, k_v, out_v, sem, dsems):
        wid = lax.axis_index("s") * NC + lax.axis_index("c")
        base = wid * b_per_w

        pltpu.sync_copy(idx_hbm.at[pl.ds(base, b_per_w)], id_v)
        mk_copies = [
            pltpu.async_copy(m_hbm.at[id_v], m_v, sem),
            pltpu.async_copy(k_hbm.at[id_v], k_v, sem),
        ]
        for g in range(b_per_w // LANES):
            sl = pl.ds(g * LANES, LANES)
            ids = id_v[sl]
            for j in range(N_CP):
                jx_vs[j][sl] = ids + jnp.int32(n_items * j)
        d_copies = [
            pltpu.async_copy(
                d_hbm.at[jx_vs[j]], d_v.at[pl.ds(j * b_per_w, b_per_w)], dsems[j]
            )
            for j in range(N_CP)
        ]
        pltpu.sync_copy(t_hbm.at[pl.ds(base, b_per_w)], t_v)
        for cp in mk_copies:
            cp.wait()

        def init_body(g, carry):
            gb = g * LANES
            out_v[pl.ds(gb, LANES)] = (
                m_v[pl.ds(gb, LANES)]
                + k_v[pl.ds(gb, LANES)] * t_v[pl.ds(gb, LANES)]
            )
            return carry

        lax.fori_loop(0, n_groups, init_body, 0)

        for j in range(N_CP):
            d_copies[j].wait()
            sj = jnp.float32(CP_STEP * (j + 1))

            def j_body(g, carry, j=j, sj=sj):
                gb = g * LANES
                tg = t_v[pl.ds(gb, LANES)]
                d = d_v[pl.ds(j * b_per_w + gb, LANES)]
                out_v[pl.ds(gb, LANES)] += jnp.where(
                    tg > sj, d * (tg - sj), 0.0
                )
                return carry

            lax.fori_loop(0, n_groups, j_body, 0)

        pltpu.sync_copy(out_v, out_hbm.at[pl.ds(base, b_per_w)])

    return sc_kernel(t, idx, m_tab, k_tab, d_flat)


def kernel(t, item_id, m_table, k_table, delta_table):
    B = t.shape[0]
    n_items = delta_table.shape[0]
    out = _trend_sc(
        t.reshape(B),
        item_id.reshape(B),
        m_table.reshape(-1),
        k_table.reshape(-1),
        delta_table.T.reshape(-1),
        b_per_w=B // NW,
        n_items=n_items,
    )
    return out.reshape(B, 1)


# speedup vs baseline: 1.0517x; 1.0517x over previous
import functools

import jax
import jax.numpy as jnp
from jax import lax
from jax.experimental import pallas as pl
from jax.experimental.pallas import tpu as pltpu
from jax.experimental.pallas import tpu_sc as plsc

N_CP = 20
CP_STEP = 40.0

NC = 2
NS = 16
NW = NC * NS
LANES = 16


@functools.partial(jax.jit, static_argnames=("b_per_w", "n_items"))
def _trend_sc(t, idx, m_tab, k_tab, d_flat, b_per_w, n_items):
    B = t.shape[0]
    n_groups = b_per_w // LANES
    mesh = plsc.VectorSubcoreMesh(core_axis_name="c", subcore_axis_name="s")

    @functools.partial(
        pl.kernel,
        mesh=mesh,
        compiler_params=pltpu.CompilerParams(
            needs_layout_passes=False, use_tc_tiling_on_sc=False
        ),
        out_type=jax.ShapeDtypeStruct((B,), jnp.float32),
        scratch_types=[
            pltpu.VMEM((b_per_w,), jnp.int32),
            [pltpu.VMEM((b_per_w,), jnp.int32)] * N_CP,
            pltpu.VMEM((N_CP * b_per_w,), jnp.float32),
            pltpu.VMEM((b_per_w,), jnp.float32),
            pltpu.VMEM((b_per_w,), jnp.float32),
            pltpu.VMEM((b_per_w,), jnp.float32),
            pltpu.VMEM((b_per_w,), jnp.float32),
            pltpu.SemaphoreType.DMA,
            [pltpu.SemaphoreType.DMA] * N_CP,
        ],
    )
    def sc_kernel(t_hbm, idx_hbm, m_hbm, k_hbm, d_hbm, out_hbm,
                  id_v, jx_vs, d_v, t_v, m_v, k_v, out_v, sem, dsems):
        wid = lax.axis_index("s") * NC + lax.axis_index("c")
        base = wid * b_per_w

        pltpu.sync_copy(idx_hbm.at[pl.ds(base, b_per_w)], id_v)
        mk_copies = [
            pltpu.async_copy(m_hbm.at[id_v], m_v, sem),
            pltpu.async_copy(k_hbm.at[id_v], k_v, sem),
        ]
        for g in range(b_per_w // LANES):
            sl = pl.ds(g * LANES, LANES)
            ids = id_v[sl]
            for j in range(N_CP):
                jx_vs[j][sl] = ids + jnp.int32(n_items * j)
        d_copies = [
            pltpu.async_copy(
                d_hbm.at[jx_vs[j]], d_v.at[pl.ds(j * b_per_w, b_per_w)], dsems[j]
            )
            for j in range(N_CP)
        ]
        pltpu.sync_copy(t_hbm.at[pl.ds(base, b_per_w)], t_v)
        for cp in mk_copies:
            cp.wait()
        for cp in d_copies:
            cp.wait()

        def body(g, carry):
            gb = g * LANES
            tg = t_v[pl.ds(gb, LANES)]
            acc = m_v[pl.ds(gb, LANES)] + k_v[pl.ds(gb, LANES)] * tg
            for j in range(N_CP):
                d = d_v[pl.ds(j * b_per_w + gb, LANES)]
                sj = jnp.float32(CP_STEP * (j + 1))
                acc += d * jnp.maximum(tg - sj, 0.0)
            out_v[pl.ds(gb, LANES)] = acc
            return carry

        lax.fori_loop(0, n_groups, body, 0)

        pltpu.sync_copy(out_v, out_hbm.at[pl.ds(base, b_per_w)])

    return sc_kernel(t, idx, m_tab, k_tab, d_flat)


def kernel(t, item_id, m_table, k_table, delta_table):
    B = t.shape[0]
    n_items = delta_table.shape[0]
    out = _trend_sc(
        t.reshape(B),
        item_id.reshape(B),
        m_table.reshape(-1),
        k_table.reshape(-1),
        delta_table.T.reshape(-1),
        b_per_w=B // NW,
        n_items=n_items,
    )
    return out.reshape(B, 1)


# speedup vs baseline: 1.0532x; 1.0014x over previous
import functools

import jax
import jax.numpy as jnp
from jax import lax
from jax.experimental import pallas as pl
from jax.experimental.pallas import tpu as pltpu
from jax.experimental.pallas import tpu_sc as plsc

N_CP = 20
CP_STEP = 40.0

NC = 2
NS = 16
NW = NC * NS
LANES = 16


@functools.partial(jax.jit, static_argnames=("b_per_w", "n_items"))
def _trend_sc(t, idx, m_tab, k_tab, d_flat, b_per_w, n_items):
    B = t.shape[0]
    n_groups = b_per_w // LANES
    mesh = plsc.VectorSubcoreMesh(core_axis_name="c", subcore_axis_name="s")

    @functools.partial(
        pl.kernel,
        mesh=mesh,
        compiler_params=pltpu.CompilerParams(
            needs_layout_passes=False, use_tc_tiling_on_sc=False
        ),
        out_type=jax.ShapeDtypeStruct((B,), jnp.float32),
        scratch_types=[
            pltpu.VMEM((b_per_w,), jnp.int32),
            pltpu.VMEM((N_CP * b_per_w,), jnp.int32),
            pltpu.VMEM((N_CP * b_per_w,), jnp.float32),
            pltpu.VMEM((b_per_w,), jnp.float32),
            pltpu.VMEM((b_per_w,), jnp.float32),
            pltpu.VMEM((b_per_w,), jnp.float32),
            pltpu.VMEM((b_per_w,), jnp.float32),
            pltpu.SemaphoreType.DMA,
            [pltpu.SemaphoreType.DMA] * N_CP,
        ],
    )
    def sc_kernel(t_hbm, idx_hbm, m_hbm, k_hbm, d_hbm, out_hbm,
                  id_v, jx_v, d_v, t_v, m_v, k_v, out_v, sem, dsems):
        wid = lax.axis_index("s") * NC + lax.axis_index("c")
        base = wid * b_per_w

        pltpu.sync_copy(idx_hbm.at[pl.ds(base, b_per_w)], id_v)
        mk_copies = [
            pltpu.async_copy(m_hbm.at[id_v], m_v, sem),
            pltpu.async_copy(k_hbm.at[id_v], k_v, sem),
        ]
        for g in range(b_per_w // LANES):
            ids = id_v[pl.ds(g * LANES, LANES)]
            for j in range(N_CP):
                jx_v[pl.ds(j * b_per_w + g * LANES, LANES)] = (
                    ids + jnp.int32(n_items * j)
                )
        d_copies = [pltpu.async_copy(d_hbm.at[jx_v], d_v, dsems[0])]
        pltpu.sync_copy(t_hbm.at[pl.ds(base, b_per_w)], t_v)
        for cp in mk_copies:
            cp.wait()
        for cp in d_copies:
            cp.wait()

        def body(g, carry):
            gb = g * LANES
            tg = t_v[pl.ds(gb, LANES)]
            acc = m_v[pl.ds(gb, LANES)] + k_v[pl.ds(gb, LANES)] * tg
            for j in range(N_CP):
                d = d_v[pl.ds(j * b_per_w + gb, LANES)]
                sj = jnp.float32(CP_STEP * (j + 1))
                acc += d * jnp.maximum(tg - sj, 0.0)
            out_v[pl.ds(gb, LANES)] = acc
            return carry

        lax.fori_loop(0, n_groups, body, 0)

        pltpu.sync_copy(out_v, out_hbm.at[pl.ds(base, b_per_w)])

    return sc_kernel(t, idx, m_tab, k_tab, d_flat)


def kernel(t, item_id, m_table, k_table, delta_table):
    B = t.shape[0]
    n_items = delta_table.shape[0]
    out = _trend_sc(
        t.reshape(B),
        item_id.reshape(B),
        m_table.reshape(-1),
        k_table.reshape(-1),
        delta_table.T.reshape(-1),
        b_per_w=B // NW,
        n_items=n_items,
    )
    return out.reshape(B, 1)
